# baseline (speedup 1.0000x reference)
import functools

import jax
import jax.numpy as jnp
from jax import lax
from jax.experimental import pallas as pl
from jax.experimental.pallas import tpu as pltpu
from jax.experimental.pallas import tpu_sc as plsc

N_SUPPORT = 16384
D = 256
NUM_CLASSES = 100
L = 16
NC = 2
NS = 16
ROWS_PER_TILE = N_SUPPORT // NS
SUB = 128
NSUB = ROWS_PER_TILE // SUB
CHUNK = 256
NCHUNK = ROWS_PER_TILE // CHUNK
DC = D // NC
CLS_PAD = 128
CPT = CLS_PAD // NS
CLS_ROWS = N_SUPPORT // SUB
FLAT_PAD = N_SUPPORT + L
BSEARCH_STEPS = 15
FULL_TILES = NUM_CLASSES // CPT
TAIL = NUM_CLASSES - FULL_TILES * CPT


def _first_ge(flat_v, c):
    def step(_, lohi):
        lo, hi = lohi
        mid = lax.div(lo + hi, jnp.int32(2))
        ge = flat_v[pl.ds(mid, L)][0] >= c
        return (jnp.where(ge, lo, mid + 1), jnp.where(ge, mid, hi))
    lo, _ = lax.fori_loop(
        0, BSEARCH_STEPS, step, (jnp.int32(0), jnp.int32(N_SUPPORT)))
    return lo


def _seg_mean_body(feat_hbm, cls2d_hbm, cls1d_hbm, out_hbm,
                   idx_v, flat_v, buf0_v, buf1_v, blk_v, acc_sh,
                   sem_f, sem_a, sem_b):
    cid = lax.axis_index("c")
    sid = lax.axis_index("s")
    col0 = cid * DC
    row0 = sid * ROWS_PER_TILE

    zeros16 = jnp.zeros((L,), jnp.float32)

    h_flat = pltpu.async_copy(cls1d_hbm, flat_v, sem_f)

    pltpu.sync_copy(cls2d_hbm.at[pl.ds(sid * NSUB, NSUB)], idx_v)

    def zrow(i, carry):
        for k in range(DC // L):
            blk_v[i, pl.ds(k * L, L)] = zeros16
        return carry
    lax.fori_loop(0, CPT, zrow, 0)
    pltpu.sync_copy(blk_v, acc_sh.at[pl.ds(sid * CPT, CPT)])

    def load_slice(k):
        return feat_hbm.at[pl.ds(row0 + k * CHUNK, CHUNK), pl.ds(col0, DC)]

    bufs = [buf0_v, buf1_v]
    sems = [sem_a, sem_b]
    hl = [pltpu.async_copy(load_slice(0), buf0_v, sem_a), None]
    plsc.subcore_barrier()

    for k in range(NCHUNK):
        b = k & 1
        hl[b].wait()
        if k + 1 < NCHUNK:
            hl[1 - b] = pltpu.async_copy(load_slice(k + 1), bufs[1 - b],
                                         sems[1 - b])
        pltpu.sync_copy(bufs[b].at[pl.ds(0, SUB)],
                        acc_sh.at[idx_v.at[2 * k]], add=True)
        pltpu.sync_copy(bufs[b].at[pl.ds(SUB, SUB)],
                        acc_sh.at[idx_v.at[2 * k + 1]], add=True)

    plsc.subcore_barrier()
    h_flat.wait()

    start = sid * CPT
    pltpu.sync_copy(acc_sh.at[pl.ds(start, CPT)], blk_v)

    def div_row(i, bound):
        nxt = _first_ge(flat_v, start + (i + 1))
        cnt = jnp.maximum(nxt - bound, 1).astype(jnp.float32)
        inv = jnp.full((L,), cnt, jnp.float32)
        for k in range(DC // L):
            blk_v[i, pl.ds(k * L, L)] = blk_v[i, pl.ds(k * L, L)] / inv
        return nxt
    lax.fori_loop(0, CPT, div_row, _first_ge(flat_v, start))

    @pl.when(sid < FULL_TILES)
    def _():
        pltpu.sync_copy(blk_v, out_hbm.at[pl.ds(start, CPT), pl.ds(col0, DC)])

    @pl.when(sid == FULL_TILES)
    def _():
        pltpu.sync_copy(blk_v.at[pl.ds(0, TAIL)],
                        out_hbm.at[pl.ds(FULL_TILES * CPT, TAIL),
                                   pl.ds(col0, DC)])


@jax.jit
def _seg_mean(support_features, cls2d, cls1d):
    mesh = plsc.VectorSubcoreMesh(core_axis_name="c", subcore_axis_name="s")
    run = functools.partial(
        pl.kernel,
        out_type=jax.ShapeDtypeStruct((NUM_CLASSES, D), jnp.float32),
        mesh=mesh,
        scratch_types=[
            pltpu.VMEM((NSUB, SUB), jnp.int32),
            pltpu.VMEM((FLAT_PAD,), jnp.int32),
            pltpu.VMEM((CHUNK, DC), jnp.float32),
            pltpu.VMEM((CHUNK, DC), jnp.float32),
            pltpu.VMEM((CPT, DC), jnp.float32),
            pltpu.VMEM_SHARED((CLS_PAD, DC), jnp.float32),
            pltpu.SemaphoreType.DMA,
            pltpu.SemaphoreType.DMA,
            pltpu.SemaphoreType.DMA,
        ],
    )(_seg_mean_body)
    return run(support_features, cls2d, cls1d)


def kernel(support_features, query_features, support_labels, query_labels):
    cls = support_labels[:, 0]
    cls2d = cls.reshape(CLS_ROWS, SUB)
    cls1d = jnp.pad(cls, (0, L), constant_values=NUM_CLASSES)
    return _seg_mean(support_features, cls2d, cls1d)
